# baseline (speedup 1.0000x reference)
import functools
import jax
import jax.numpy as jnp
from jax import lax
from jax.experimental import pallas as pl
from jax.experimental.pallas import tpu as pltpu
from jax.experimental.pallas import tpu_sc as plsc

F32 = jnp.float32
I32 = jnp.int32

NC, NS, LANES = 2, 16, 16
NW = NC * NS
CHUNK = 640
SUB = 128
NSUB = CHUNK // SUB
GPS = SUB // LANES

_CP = pltpu.CompilerParams(needs_layout_passes=False,
                           use_tc_tiling_on_sc=False)


def _mesh():
    return plsc.VectorSubcoreMesh(
        core_axis_name="c", subcore_axis_name="s",
        num_cores=NC, num_subcores=NS)


def _splat(v, j):
    return lax.gather(
        v, jnp.full((LANES, 1), j, I32),
        dimension_numbers=lax.GatherDimensionNumbers(
            offset_dims=(), collapsed_slice_dims=(0,), start_index_map=(0,)),
        slice_sizes=(1,), mode=lax.GatherScatterMode.PROMISE_IN_BOUNDS)


def _make_score(N, E):
    nchunks = E // CHUNK

    @functools.partial(
        pl.kernel,
        out_type=(jax.ShapeDtypeStruct((E,), F32),
                  jax.ShapeDtypeStruct((E,), F32),
                  jax.ShapeDtypeStruct((NW, LANES), F32),
                  jax.ShapeDtypeStruct((NW, LANES), F32)),
        mesh=_mesh(),
        compiler_params=_CP,
        scratch_types=[
            pltpu.VMEM((N,), F32),
            pltpu.VMEM((N,), F32),
            pltpu.VMEM((NS, LANES), F32),
            pltpu.VMEM((CHUNK,), I32),
            pltpu.VMEM((CHUNK,), I32),
            pltpu.VMEM((CHUNK,), F32),
            pltpu.VMEM((CHUNK,), F32),
            pltpu.VMEM((CHUNK,), F32),
            pltpu.VMEM((LANES,), F32),
            pltpu.SemaphoreType.DMA,
        ])
    def score(row_h, col_h, a_h, b_h, ew_h, s_out_h, c_out_h, m_h, z_h,
              a_v, b_v, mrd_v, row_v, col_v, s_v, ew_v, c_v, st_v, sem):
        cid = lax.axis_index("c")
        sid = lax.axis_index("s")
        wid = cid * NS + sid
        nk = (nchunks - wid + NW - 1) // NW
        pltpu.sync_copy(a_h, a_v)
        pltpu.sync_copy(b_h, b_v)

        def score_chunk(i, m_acc):
            off = (wid + i * NW) * CHUNK
            cp1 = pltpu.async_copy(row_h.at[pl.ds(off, CHUNK)], row_v, sem)
            cp2 = pltpu.async_copy(col_h.at[pl.ds(off, CHUNK)], col_v, sem)
            cp1.wait()
            cp2.wait()

            def grp(g, m):
                o = g * LANES
                r = row_v[pl.ds(o, LANES)]
                c = col_v[pl.ds(o, LANES)]
                av = plsc.load_gather(a_v, [r])
                bv = plsc.load_gather(b_v, [c])
                s0 = av + bv
                s = jnp.where(s0 >= 0.0, s0, 0.01 * s0)
                s_v[pl.ds(o, LANES)] = s
                return jnp.maximum(m, s)

            m_acc = lax.fori_loop(0, CHUNK // LANES, grp, m_acc)
            pltpu.sync_copy(s_v, s_out_h.at[pl.ds(off, CHUNK)])
            return m_acc

        m_acc = lax.fori_loop(0, nk, score_chunk,
                              jnp.full((LANES,), -jnp.inf, F32))
        st_v[...] = m_acc
        pltpu.sync_copy(st_v, m_h.at[wid])
        plsc.subcore_barrier()
        pltpu.sync_copy(m_h.at[pl.ds(cid * NS, NS)], mrd_v)
        macc = mrd_v[0, pl.ds(0, LANES)]
        for k in range(1, NS):
            macc = jnp.maximum(macc, mrd_v[k, pl.ds(0, LANES)])
        mc = jnp.max(macc)
        mv = jnp.full((LANES,), mc, F32)

        def coef_chunk(i, z_acc):
            off = (wid + i * NW) * CHUNK
            cp1 = pltpu.async_copy(s_out_h.at[pl.ds(off, CHUNK)], s_v, sem)
            cp2 = pltpu.async_copy(ew_h.at[pl.ds(off, CHUNK)], ew_v, sem)
            cp1.wait()
            cp2.wait()

            def grp(g, z):
                o = g * LANES
                ex = jnp.exp(s_v[pl.ds(o, LANES)] - mv)
                c_v[pl.ds(o, LANES)] = ex * ew_v[pl.ds(o, LANES)]
                return z + ex

            z_acc = lax.fori_loop(0, CHUNK // LANES, grp, z_acc)
            pltpu.sync_copy(c_v, c_out_h.at[pl.ds(off, CHUNK)])
            return z_acc

        z_acc = lax.fori_loop(0, nk, coef_chunk, jnp.zeros((LANES,), F32))
        st_v[...] = z_acc
        pltpu.sync_copy(st_v, z_h.at[wid])

    return score


def _make_message(N, E, DH, np_, subr, last_sz):
    nq = DH // LANES
    nchunks = E // CHUNK
    zk, zt = divmod(subr, 128)

    @functools.partial(
        pl.kernel,
        out_type=jax.ShapeDtypeStruct((NC, nq, np_, LANES), F32),
        mesh=_mesh(),
        compiler_params=_CP,
        scratch_types=[
            pltpu.VMEM_SHARED((np_, LANES), F32),
            pltpu.VMEM_SHARED((np_, LANES), F32),
            pltpu.VMEM((NSUB, SUB), I32),
            pltpu.VMEM((NSUB, SUB), I32),
            pltpu.VMEM((CHUNK,), F32),
            pltpu.VMEM((CHUNK, LANES), F32),
            pltpu.VMEM((128, LANES), F32),
            pltpu.SemaphoreType.DMA,
        ])
    def message(row2_h, col2_h, c_h, hq0, hq1, hq2, hq3, parts_h,
                tab_sh, agg_sh, row2d_v, col2d_v, c_v, rows_v, zbuf, sem):
        cid = lax.axis_index("c")
        sid = lax.axis_index("s")
        wid = cid * NS + sid
        nk = (nchunks - wid + NW - 1) // NW
        r0 = sid * subr
        hqs = [hq0, hq1, hq2, hq3]

        def zrow(j, _):
            zbuf[j, pl.ds(0, LANES)] = jnp.zeros((LANES,), F32)
            return 0

        lax.fori_loop(0, 128, zrow, 0)

        for q in range(nq):
            @pl.when(sid < NS - 1)
            def _(q=q):
                pltpu.sync_copy(hqs[q].at[pl.ds(r0, subr)],
                                tab_sh.at[pl.ds(r0, subr)])

            @pl.when(sid == NS - 1)
            def _(q=q):
                pltpu.sync_copy(hqs[q].at[pl.ds((NS - 1) * subr, last_sz)],
                                tab_sh.at[pl.ds((NS - 1) * subr, last_sz)])

            for kk in range(zk):
                pltpu.sync_copy(zbuf, agg_sh.at[pl.ds(r0 + kk * 128, 128)])
            if zt:
                pltpu.sync_copy(zbuf.at[pl.ds(0, zt)],
                                agg_sh.at[pl.ds(r0 + zk * 128, zt)])
            plsc.subcore_barrier()

            def chunk(i, _):
                off = (wid + i * NW) * CHUNK
                offr = (wid + i * NW) * NSUB
                cps = [pltpu.async_copy(
                    row2_h.at[pl.ds(offr, NSUB)], row2d_v, sem),
                    pltpu.async_copy(
                        col2_h.at[pl.ds(offr, NSUB)], col2d_v, sem),
                    pltpu.async_copy(
                        c_h.at[pl.ds(off, CHUNK)], c_v, sem)]
                for cp in cps:
                    cp.wait()
                gps_ = [pltpu.async_copy(
                    tab_sh.at[col2d_v.at[b]],
                    rows_v.at[pl.ds(b * SUB, SUB)], sem)
                    for b in range(NSUB)]
                for cp in gps_:
                    cp.wait()

                def edge16(eb, _):
                    e0 = eb * LANES
                    c16 = c_v[pl.ds(e0, LANES)]
                    for j in range(LANES):
                        cs = _splat(c16, j)
                        rows_v[e0 + j, pl.ds(0, LANES)] = (
                            rows_v[e0 + j, pl.ds(0, LANES)] * cs)
                    return 0

                lax.fori_loop(0, CHUNK // LANES, edge16, 0)
                sps = [pltpu.async_copy(rows_v.at[pl.ds(b * SUB, SUB)],
                                        agg_sh.at[row2d_v.at[b]], sem,
                                        add=True)
                       for b in range(NSUB)]
                for cp in sps:
                    cp.wait()
                return 0

            lax.fori_loop(0, nk, chunk, 0)
            plsc.subcore_barrier()
            for kk in range(zk):
                pltpu.sync_copy(
                    agg_sh.at[pl.ds(r0 + kk * 128, 128)],
                    parts_h.at[cid, q, pl.ds(r0 + kk * 128, 128)])
            if zt:
                pltpu.sync_copy(
                    agg_sh.at[pl.ds(r0 + zk * 128, zt)],
                    parts_h.at[cid, q, pl.ds(r0 + zk * 128, zt)])
            plsc.subcore_barrier()

    return message


def _make_pre(N, DIN, DH, rb, with_x):
    nblk = N // rb
    nq = DH // LANES
    full = lambda shape: pl.BlockSpec(shape, lambda i: (0,) * len(shape))
    in_specs = []
    if with_x:
        in_specs += [pl.BlockSpec((rb, DIN), lambda i: (i, 0)),
                     full((DIN, DH)), full((1, DH))]
    else:
        in_specs += [pl.BlockSpec((rb, DH), lambda i: (i, 0))]
    in_specs += [full((DH, DH)), full((1, DH)),
                 full((DH, 1)), full((DH, 1)), full((1, 1))]

    outs = ([jax.ShapeDtypeStruct((N, DH), F32)] if with_x else [])
    outs += [jax.ShapeDtypeStruct((N, LANES), F32) for _ in range(nq)]
    outs += [jax.ShapeDtypeStruct((N, 1), F32),
             jax.ShapeDtypeStruct((N, 1), F32)]
    out_specs = ([pl.BlockSpec((rb, DH), lambda i: (i, 0))] if with_x else [])
    out_specs += [pl.BlockSpec((rb, LANES), lambda i: (i, 0))
                  for _ in range(nq)]
    out_specs += [pl.BlockSpec((rb, 1), lambda i: (i, 0)),
                  pl.BlockSpec((rb, 1), lambda i: (i, 0))]

    def body(*refs):
        if with_x:
            (x_r, wi_r, bi_r, w_r, b_r, wai_r, waj_r, ba_r,
             h_r, *rest) = refs
            h = jnp.dot(x_r[...], wi_r[...],
                        preferred_element_type=F32) + bi_r[...]
            h_r[...] = h
        else:
            (h_ref, w_r, b_r, wai_r, waj_r, ba_r, *rest) = refs
            h = h_ref[...]
        hq_refs = rest[:nq]
        a_r, bv_r = rest[nq], rest[nq + 1]
        hl = jnp.dot(h, w_r[...], preferred_element_type=F32) + b_r[...]
        for q in range(nq):
            hq_refs[q][...] = hl[:, q * LANES:(q + 1) * LANES]
        a_r[...] = jnp.dot(hl, wai_r[...],
                           preferred_element_type=F32) + ba_r[...]
        bv_r[...] = jnp.dot(hl, waj_r[...], preferred_element_type=F32)

    return pl.pallas_call(
        body, grid=(nblk,), in_specs=in_specs,
        out_specs=out_specs, out_shape=outs)


def _make_post(N, DH, rb, head):
    nblk = N // rb
    nq = DH // LANES
    full = lambda shape: pl.BlockSpec(shape, lambda i: (0,) * len(shape))
    in_specs = [pl.BlockSpec((rb, DH), lambda i: (i, 0))]
    in_specs += [pl.BlockSpec((rb, LANES), lambda i: (i, 0))
                 for _ in range(nq)]
    for c in range(NC):
        for q in range(nq):
            in_specs.append(pl.BlockSpec(
                (1, 1, rb, LANES),
                lambda i, c=c, q=q: (c, q, i, 0)))
    in_specs += [full((1, DH)), full((1, DH)),
                 full((1, 1)), full((1, 1))]
    if head:
        dmid = DH // 2
        in_specs += [full((DH, dmid)), full((1, dmid)),
                     full((dmid, 1)), full((1, 1))]
        out_shape = jax.ShapeDtypeStruct((N, 1), F32)
        out_spec = pl.BlockSpec((rb, 1), lambda i: (i, 0))
    else:
        out_shape = jax.ShapeDtypeStruct((N, DH), F32)
        out_spec = pl.BlockSpec((rb, DH), lambda i: (i, 0))

    def body(*refs):
        h_r = refs[0]
        hq_refs = refs[1:1 + nq]
        p_refs = refs[1 + nq:1 + nq + NC * nq]
        g_r, be_r, w0_r, w1_r = refs[1 + nq + NC * nq:5 + nq + NC * nq]
        out_r = refs[-1]
        w0 = w0_r[0, 0]
        w1 = w1_r[0, 0]
        cols = []
        for q in range(nq):
            agg = p_refs[q][0, 0] * w0 + p_refs[nq + q][0, 0] * w1
            cols.append(hq_refs[q][...] + agg)
        o = jnp.maximum(jnp.concatenate(cols, axis=-1), 0.0)
        mu = jnp.mean(o, axis=-1, keepdims=True)
        d = o - mu
        var = jnp.mean(d * d, axis=-1, keepdims=True)
        hn = d * lax.rsqrt(var + 1e-5) * g_r[...] + be_r[...]
        hnew = h_r[...] + hn
        if head:
            wo1_r, bo1_r, wo2_r, bo2_r = refs[5 + nq + NC * nq:-1]
            y = jnp.maximum(
                jnp.dot(hnew, wo1_r[...],
                        preferred_element_type=F32) + bo1_r[...], 0.0)
            y = jnp.dot(y, wo2_r[...],
                        preferred_element_type=F32) + bo2_r[...]
            out_r[...] = jax.nn.sigmoid(y)
        else:
            out_r[...] = hnew

    return pl.pallas_call(
        body, grid=(nblk,), in_specs=in_specs,
        out_specs=out_spec, out_shape=out_shape)


def _make_mid(N, DH, rb):
    nblk = N // rb
    nq = DH // LANES
    full = lambda shape: pl.BlockSpec(shape, lambda i: (0,) * len(shape))
    in_specs = [pl.BlockSpec((rb, DH), lambda i: (i, 0))]
    in_specs += [pl.BlockSpec((rb, LANES), lambda i: (i, 0))
                 for _ in range(nq)]
    for c in range(NC):
        for q in range(nq):
            in_specs.append(pl.BlockSpec(
                (1, 1, rb, LANES),
                lambda i, c=c, q=q: (c, q, i, 0)))
    in_specs += [full((1, DH)), full((1, DH)),
                 full((1, 1)), full((1, 1))]
    in_specs += [full((DH, DH)), full((1, DH)),
                 full((DH, 1)), full((DH, 1)), full((1, 1))]

    outs = [jax.ShapeDtypeStruct((N, DH), F32)]
    outs += [jax.ShapeDtypeStruct((N, LANES), F32) for _ in range(nq)]
    outs += [jax.ShapeDtypeStruct((N, 1), F32),
             jax.ShapeDtypeStruct((N, 1), F32)]
    out_specs = [pl.BlockSpec((rb, DH), lambda i: (i, 0))]
    out_specs += [pl.BlockSpec((rb, LANES), lambda i: (i, 0))
                  for _ in range(nq)]
    out_specs += [pl.BlockSpec((rb, 1), lambda i: (i, 0)),
                  pl.BlockSpec((rb, 1), lambda i: (i, 0))]

    def body(*refs):
        h_r = refs[0]
        hq_refs = refs[1:1 + nq]
        p_refs = refs[1 + nq:1 + nq + NC * nq]
        (g_r, be_r, w0_r, w1_r, w_r, b_r, wai_r, waj_r, ba_r,
         hn_r, *rest) = refs[1 + nq + NC * nq:]
        hqo_refs = rest[:nq]
        a_r, bv_r = rest[nq], rest[nq + 1]
        w0 = w0_r[0, 0]
        w1 = w1_r[0, 0]
        cols = []
        for q in range(nq):
            agg = p_refs[q][0, 0] * w0 + p_refs[nq + q][0, 0] * w1
            cols.append(hq_refs[q][...] + agg)
        o = jnp.maximum(jnp.concatenate(cols, axis=-1), 0.0)
        mu = jnp.mean(o, axis=-1, keepdims=True)
        d = o - mu
        var = jnp.mean(d * d, axis=-1, keepdims=True)
        hn = d * lax.rsqrt(var + 1e-5) * g_r[...] + be_r[...]
        hnew = h_r[...] + hn
        hn_r[...] = hnew
        hl = jnp.dot(hnew, w_r[...], preferred_element_type=F32) + b_r[...]
        for q in range(nq):
            hqo_refs[q][...] = hl[:, q * LANES:(q + 1) * LANES]
        a_r[...] = jnp.dot(hl, wai_r[...],
                           preferred_element_type=F32) + ba_r[...]
        bv_r[...] = jnp.dot(hl, waj_r[...], preferred_element_type=F32)

    return pl.pallas_call(
        body, grid=(nblk,), in_specs=in_specs,
        out_specs=out_specs, out_shape=outs)


def kernel(x, edge_index, edge_weight, Wi, bi,
           W0, b0, Wa0, ba0, g0, be0,
           W1, b1, Wa1, ba1, g1, be1,
           W2, b2, Wa2, ba2, g2, be2,
           Wo1, bo1, Wo2, bo2):
    N, DIN = x.shape
    E = edge_index.shape[1]
    DH = Wi.shape[1]
    nq = DH // LANES
    assert E % CHUNK == 0 and DH % LANES == 0
    np_ = -(-N // (8 * NS)) * (8 * NS)
    subr = np_ // NS
    last_sz = N - (NS - 1) * subr
    assert last_sz > 0 and last_sz % 8 == 0
    rb = 400
    assert N % rb == 0

    row = edge_index[0]
    col = edge_index[1]

    pre0 = _make_pre(N, DIN, DH, rb, True)
    score = _make_score(N, E)
    message = _make_message(N, E, DH, np_, subr, last_sz)
    mid = _make_mid(N, DH, rb)
    post_head = _make_post(N, DH, rb, True)

    layers = [(W0, b0, Wa0, ba0, g0, be0),
              (W1, b1, Wa1, ba1, g1, be1),
              (W2, b2, Wa2, ba2, g2, be2)]

    def edge_phase(a, bv):
        _s, c, m, z = score(row, col, a.reshape(N), bv.reshape(N),
                            edge_weight)
        parts = message(row.reshape(-1, SUB), col.reshape(-1, SUB),
                        c, *hqs)
        mr = m.reshape(NC, NS * LANES).max(axis=1)
        M = jnp.maximum(mr[0], mr[1])
        zr = z.reshape(NC, NS * LANES).sum(axis=1)
        e0 = jnp.exp(mr[0] - M)
        e1 = jnp.exp(mr[1] - M)
        Z = e0 * zr[0] + e1 * zr[1]
        return parts, (e0 / Z).reshape(1, 1), (e1 / Z).reshape(1, 1)

    W, b, Wa, ba, g, be = layers[0]
    h, *hqs, a, bv = pre0(x, Wi, bi.reshape(1, DH), W, b.reshape(1, DH),
                          Wa[:DH], Wa[DH:], ba.reshape(1, 1))
    for l in range(2):
        parts, w0, w1 = edge_phase(a, bv)
        Wn, bn, Wan, ban, gn, ben = layers[l + 1]
        args = [h] + hqs + [parts] * (NC * nq) + [
            g.reshape(1, DH), be.reshape(1, DH), w0, w1,
            Wn, bn.reshape(1, DH), Wan[:DH], Wan[DH:], ban.reshape(1, 1)]
        h, *rest = mid(*args)
        hqs = rest[:nq]
        a, bv = rest[nq], rest[nq + 1]
        g, be = gn, ben
    parts, w0, w1 = edge_phase(a, bv)
    args = [h] + hqs + [parts] * (NC * nq) + [
        g.reshape(1, DH), be.reshape(1, DH), w0, w1,
        Wo1, bo1.reshape(1, DH // 2), Wo2, bo2.reshape(1, 1)]
    return post_head(*args)


# speedup vs baseline: 1.1643x; 1.1643x over previous
import functools
import jax
import jax.numpy as jnp
from jax import lax
from jax.experimental import pallas as pl
from jax.experimental.pallas import tpu as pltpu
from jax.experimental.pallas import tpu_sc as plsc

F32 = jnp.float32
I32 = jnp.int32

NC, NS, LANES = 2, 16, 16
NW = NC * NS
CHUNK = 640
SUB = 128
NSUB = CHUNK // SUB
GPS = SUB // LANES

_CP = pltpu.CompilerParams(needs_layout_passes=False,
                           use_tc_tiling_on_sc=False)


def _mesh():
    return plsc.VectorSubcoreMesh(
        core_axis_name="c", subcore_axis_name="s",
        num_cores=NC, num_subcores=NS)


def _splat(v, j):
    return lax.gather(
        v, jnp.full((LANES, 1), j, I32),
        dimension_numbers=lax.GatherDimensionNumbers(
            offset_dims=(), collapsed_slice_dims=(0,), start_index_map=(0,)),
        slice_sizes=(1,), mode=lax.GatherScatterMode.PROMISE_IN_BOUNDS)


def _make_score(N, E):
    nchunks = E // CHUNK

    @functools.partial(
        pl.kernel,
        out_type=(jax.ShapeDtypeStruct((E,), F32),
                  jax.ShapeDtypeStruct((E,), F32),
                  jax.ShapeDtypeStruct((NW, LANES), F32),
                  jax.ShapeDtypeStruct((NW, LANES), F32)),
        mesh=_mesh(),
        compiler_params=_CP,
        scratch_types=[
            pltpu.VMEM((N,), F32),
            pltpu.VMEM((N,), F32),
            pltpu.VMEM((NS, LANES), F32),
            pltpu.VMEM((CHUNK,), I32),
            pltpu.VMEM((CHUNK,), I32),
            pltpu.VMEM((CHUNK,), F32),
            pltpu.VMEM((CHUNK,), F32),
            pltpu.VMEM((CHUNK,), F32),
            pltpu.VMEM((LANES,), F32),
            pltpu.SemaphoreType.DMA,
        ])
    def score(row_h, col_h, a_h, b_h, ew_h, s_out_h, c_out_h, m_h, z_h,
              a_v, b_v, mrd_v, row_v, col_v, s_v, ew_v, c_v, st_v, sem):
        cid = lax.axis_index("c")
        sid = lax.axis_index("s")
        wid = cid * NS + sid
        nk = (nchunks - wid + NW - 1) // NW
        pltpu.sync_copy(a_h, a_v)
        pltpu.sync_copy(b_h, b_v)

        def score_chunk(i, m_acc):
            off = (wid + i * NW) * CHUNK
            cp1 = pltpu.async_copy(row_h.at[pl.ds(off, CHUNK)], row_v, sem)
            cp2 = pltpu.async_copy(col_h.at[pl.ds(off, CHUNK)], col_v, sem)
            cp1.wait()
            cp2.wait()

            def grp(g, m):
                o = g * LANES
                r = row_v[pl.ds(o, LANES)]
                c = col_v[pl.ds(o, LANES)]
                av = plsc.load_gather(a_v, [r])
                bv = plsc.load_gather(b_v, [c])
                s0 = av + bv
                s = jnp.where(s0 >= 0.0, s0, 0.01 * s0)
                s_v[pl.ds(o, LANES)] = s
                return jnp.maximum(m, s)

            m_acc = lax.fori_loop(0, CHUNK // LANES, grp, m_acc)
            pltpu.sync_copy(s_v, s_out_h.at[pl.ds(off, CHUNK)])
            return m_acc

        m_acc = lax.fori_loop(0, nk, score_chunk,
                              jnp.full((LANES,), -jnp.inf, F32))
        st_v[...] = m_acc
        pltpu.sync_copy(st_v, m_h.at[wid])
        plsc.subcore_barrier()
        pltpu.sync_copy(m_h.at[pl.ds(cid * NS, NS)], mrd_v)
        macc = mrd_v[0, pl.ds(0, LANES)]
        for k in range(1, NS):
            macc = jnp.maximum(macc, mrd_v[k, pl.ds(0, LANES)])
        mc = jnp.max(macc)
        mv = jnp.full((LANES,), mc, F32)

        def coef_chunk(i, z_acc):
            off = (wid + i * NW) * CHUNK
            cp1 = pltpu.async_copy(s_out_h.at[pl.ds(off, CHUNK)], s_v, sem)
            cp2 = pltpu.async_copy(ew_h.at[pl.ds(off, CHUNK)], ew_v, sem)
            cp1.wait()
            cp2.wait()

            def grp(g, z):
                o = g * LANES
                ex = jnp.exp(s_v[pl.ds(o, LANES)] - mv)
                c_v[pl.ds(o, LANES)] = ex * ew_v[pl.ds(o, LANES)]
                return z + ex

            z_acc = lax.fori_loop(0, CHUNK // LANES, grp, z_acc)
            pltpu.sync_copy(c_v, c_out_h.at[pl.ds(off, CHUNK)])
            return z_acc

        z_acc = lax.fori_loop(0, nk, coef_chunk, jnp.zeros((LANES,), F32))
        st_v[...] = z_acc
        pltpu.sync_copy(st_v, z_h.at[wid])

    return score


def _make_message(N, E, DH, np_, subr, last_sz):
    nq = DH // LANES
    nchunks = E // CHUNK
    zk, zt = divmod(subr, 128)

    @functools.partial(
        pl.kernel,
        out_type=jax.ShapeDtypeStruct((NC, nq, np_, LANES), F32),
        mesh=_mesh(),
        compiler_params=_CP,
        scratch_types=[
            pltpu.VMEM_SHARED((np_, LANES), F32),
            pltpu.VMEM_SHARED((np_, LANES), F32),
            pltpu.VMEM((NSUB, SUB), I32),
            pltpu.VMEM((NSUB, SUB), I32),
            pltpu.VMEM((CHUNK,), F32),
            pltpu.VMEM((CHUNK, LANES), F32),
            pltpu.VMEM((NSUB, SUB), I32),
            pltpu.VMEM((NSUB, SUB), I32),
            pltpu.VMEM((CHUNK,), F32),
            pltpu.VMEM((CHUNK, LANES), F32),
            pltpu.VMEM((128, LANES), F32),
            pltpu.SemaphoreType.DMA,
            pltpu.SemaphoreType.DMA,
            pltpu.SemaphoreType.DMA,
            pltpu.SemaphoreType.DMA,
        ])
    def message(row_h, col_h, c_h, hq0, hq1, hq2, hq3, parts_h,
                tab_sh, agg_sh, row2d_a, col2d_a, c_a, rows_a,
                row2d_b, col2d_b, c_b, rows_b, zbuf,
                sem_ia, sem_ib, sem_g, sem_s):
        cid = lax.axis_index("c")
        sid = lax.axis_index("s")
        wid = cid * NS + sid
        nk = (nchunks - wid + NW - 1) // NW
        r0 = sid * subr
        hqs = [hq0, hq1, hq2, hq3]

        def zrow(j, _):
            zbuf[j, pl.ds(0, LANES)] = jnp.zeros((LANES,), F32)
            return 0

        lax.fori_loop(0, 128, zrow, 0)

        def fire_inputs(k, row2d_v, col2d_v, c_v, sem):
            off = k * CHUNK
            cps = []
            for b in range(NSUB):
                cps.append(pltpu.async_copy(
                    row_h.at[pl.ds(off + b * SUB, SUB)],
                    row2d_v.at[b], sem))
                cps.append(pltpu.async_copy(
                    col_h.at[pl.ds(off + b * SUB, SUB)],
                    col2d_v.at[b], sem))
            cps.append(pltpu.async_copy(
                c_h.at[pl.ds(off, CHUNK)], c_v, sem))
            return cps

        def fire_gathers(col2d_v, rows_v):
            return [pltpu.async_copy(
                tab_sh.at[col2d_v.at[b]],
                rows_v.at[pl.ds(b * SUB, SUB)], sem_g)
                for b in range(NSUB)]

        def multiply(c_v, rows_v):
            def edge16(eb, _):
                e0 = eb * LANES
                c16 = c_v[pl.ds(e0, LANES)]
                for j in range(LANES):
                    cs = _splat(c16, j)
                    rows_v[e0 + j, pl.ds(0, LANES)] = (
                        rows_v[e0 + j, pl.ds(0, LANES)] * cs)
                return 0

            lax.fori_loop(0, CHUNK // LANES, edge16, 0)

        def fire_scatters(row2d_v, rows_v):
            return [pltpu.async_copy(rows_v.at[pl.ds(b * SUB, SUB)],
                                     agg_sh.at[row2d_v.at[b]], sem_s,
                                     add=True)
                    for b in range(NSUB)]

        for q in range(nq):
            @pl.when(sid < NS - 1)
            def _(q=q):
                pltpu.sync_copy(hqs[q].at[pl.ds(r0, subr)],
                                tab_sh.at[pl.ds(r0, subr)])

            @pl.when(sid == NS - 1)
            def _(q=q):
                pltpu.sync_copy(hqs[q].at[pl.ds((NS - 1) * subr, last_sz)],
                                tab_sh.at[pl.ds((NS - 1) * subr, last_sz)])

            for kk in range(zk):
                pltpu.sync_copy(zbuf, agg_sh.at[pl.ds(r0 + kk * 128, 128)])
            if zt:
                pltpu.sync_copy(zbuf.at[pl.ds(0, zt)],
                                agg_sh.at[pl.ds(r0 + zk * 128, zt)])
            plsc.subcore_barrier()

            def pair(i, _):
                k0 = wid + (2 * i) * NW
                k1 = wid + (2 * i + 1) * NW
                cps0 = fire_inputs(k0, row2d_a, col2d_a, c_a, sem_ia)
                cps1 = fire_inputs(k1, row2d_b, col2d_b, c_b, sem_ib)
                for cp in cps0:
                    cp.wait()
                g0 = fire_gathers(col2d_a, rows_a)
                for cp in g0:
                    cp.wait()
                for cp in cps1:
                    cp.wait()
                g1 = fire_gathers(col2d_b, rows_b)
                multiply(c_a, rows_a)
                s0 = fire_scatters(row2d_a, rows_a)
                for cp in g1:
                    cp.wait()
                multiply(c_b, rows_b)
                s1 = fire_scatters(row2d_b, rows_b)
                for cp in s0 + s1:
                    cp.wait()
                return 0

            lax.fori_loop(0, nk // 2, pair, 0)

            @pl.when(nk % 2 == 1)
            def _():
                k0 = wid + (nk - 1) * NW
                cps0 = fire_inputs(k0, row2d_a, col2d_a, c_a, sem_ia)
                for cp in cps0:
                    cp.wait()
                g0 = fire_gathers(col2d_a, rows_a)
                for cp in g0:
                    cp.wait()
                multiply(c_a, rows_a)
                s0 = fire_scatters(row2d_a, rows_a)
                for cp in s0:
                    cp.wait()

            plsc.subcore_barrier()
            for kk in range(zk):
                pltpu.sync_copy(
                    agg_sh.at[pl.ds(r0 + kk * 128, 128)],
                    parts_h.at[cid, q, pl.ds(r0 + kk * 128, 128)])
            if zt:
                pltpu.sync_copy(
                    agg_sh.at[pl.ds(r0 + zk * 128, zt)],
                    parts_h.at[cid, q, pl.ds(r0 + zk * 128, zt)])
            plsc.subcore_barrier()

    return message


def _make_pre(N, DIN, DH, rb, with_x):
    nblk = N // rb
    nq = DH // LANES
    full = lambda shape: pl.BlockSpec(shape, lambda i: (0,) * len(shape))
    in_specs = []
    if with_x:
        in_specs += [pl.BlockSpec((rb, DIN), lambda i: (i, 0)),
                     full((DIN, DH)), full((1, DH))]
    else:
        in_specs += [pl.BlockSpec((rb, DH), lambda i: (i, 0))]
    in_specs += [full((DH, DH)), full((1, DH)),
                 full((DH, 1)), full((DH, 1)), full((1, 1))]

    outs = ([jax.ShapeDtypeStruct((N, DH), F32)] if with_x else [])
    outs += [jax.ShapeDtypeStruct((N, LANES), F32) for _ in range(nq)]
    outs += [jax.ShapeDtypeStruct((N, 1), F32),
             jax.ShapeDtypeStruct((N, 1), F32)]
    out_specs = ([pl.BlockSpec((rb, DH), lambda i: (i, 0))] if with_x else [])
    out_specs += [pl.BlockSpec((rb, LANES), lambda i: (i, 0))
                  for _ in range(nq)]
    out_specs += [pl.BlockSpec((rb, 1), lambda i: (i, 0)),
                  pl.BlockSpec((rb, 1), lambda i: (i, 0))]

    def body(*refs):
        if with_x:
            (x_r, wi_r, bi_r, w_r, b_r, wai_r, waj_r, ba_r,
             h_r, *rest) = refs
            h = jnp.dot(x_r[...], wi_r[...],
                        preferred_element_type=F32) + bi_r[...]
            h_r[...] = h
        else:
            (h_ref, w_r, b_r, wai_r, waj_r, ba_r, *rest) = refs
            h = h_ref[...]
        hq_refs = rest[:nq]
        a_r, bv_r = rest[nq], rest[nq + 1]
        hl = jnp.dot(h, w_r[...], preferred_element_type=F32) + b_r[...]
        for q in range(nq):
            hq_refs[q][...] = hl[:, q * LANES:(q + 1) * LANES]
        a_r[...] = jnp.dot(hl, wai_r[...],
                           preferred_element_type=F32) + ba_r[...]
        bv_r[...] = jnp.dot(hl, waj_r[...], preferred_element_type=F32)

    return pl.pallas_call(
        body, grid=(nblk,), in_specs=in_specs,
        out_specs=out_specs, out_shape=outs)


def _make_post(N, DH, rb, head):
    nblk = N // rb
    nq = DH // LANES
    full = lambda shape: pl.BlockSpec(shape, lambda i: (0,) * len(shape))
    in_specs = [pl.BlockSpec((rb, DH), lambda i: (i, 0))]
    in_specs += [pl.BlockSpec((rb, LANES), lambda i: (i, 0))
                 for _ in range(nq)]
    for c in range(NC):
        for q in range(nq):
            in_specs.append(pl.BlockSpec(
                (1, 1, rb, LANES),
                lambda i, c=c, q=q: (c, q, i, 0)))
    in_specs += [full((1, DH)), full((1, DH)),
                 full((1, 1)), full((1, 1))]
    if head:
        dmid = DH // 2
        in_specs += [full((DH, dmid)), full((1, dmid)),
                     full((dmid, 1)), full((1, 1))]
        out_shape = jax.ShapeDtypeStruct((N, 1), F32)
        out_spec = pl.BlockSpec((rb, 1), lambda i: (i, 0))
    else:
        out_shape = jax.ShapeDtypeStruct((N, DH), F32)
        out_spec = pl.BlockSpec((rb, DH), lambda i: (i, 0))

    def body(*refs):
        h_r = refs[0]
        hq_refs = refs[1:1 + nq]
        p_refs = refs[1 + nq:1 + nq + NC * nq]
        g_r, be_r, w0_r, w1_r = refs[1 + nq + NC * nq:5 + nq + NC * nq]
        out_r = refs[-1]
        w0 = w0_r[0, 0]
        w1 = w1_r[0, 0]
        cols = []
        for q in range(nq):
            agg = p_refs[q][0, 0] * w0 + p_refs[nq + q][0, 0] * w1
            cols.append(hq_refs[q][...] + agg)
        o = jnp.maximum(jnp.concatenate(cols, axis=-1), 0.0)
        mu = jnp.mean(o, axis=-1, keepdims=True)
        d = o - mu
        var = jnp.mean(d * d, axis=-1, keepdims=True)
        hn = d * lax.rsqrt(var + 1e-5) * g_r[...] + be_r[...]
        hnew = h_r[...] + hn
        if head:
            wo1_r, bo1_r, wo2_r, bo2_r = refs[5 + nq + NC * nq:-1]
            y = jnp.maximum(
                jnp.dot(hnew, wo1_r[...],
                        preferred_element_type=F32) + bo1_r[...], 0.0)
            y = jnp.dot(y, wo2_r[...],
                        preferred_element_type=F32) + bo2_r[...]
            out_r[...] = jax.nn.sigmoid(y)
        else:
            out_r[...] = hnew

    return pl.pallas_call(
        body, grid=(nblk,), in_specs=in_specs,
        out_specs=out_spec, out_shape=out_shape)


def _make_mid(N, DH, rb):
    nblk = N // rb
    nq = DH // LANES
    full = lambda shape: pl.BlockSpec(shape, lambda i: (0,) * len(shape))
    in_specs = [pl.BlockSpec((rb, DH), lambda i: (i, 0))]
    in_specs += [pl.BlockSpec((rb, LANES), lambda i: (i, 0))
                 for _ in range(nq)]
    for c in range(NC):
        for q in range(nq):
            in_specs.append(pl.BlockSpec(
                (1, 1, rb, LANES),
                lambda i, c=c, q=q: (c, q, i, 0)))
    in_specs += [full((1, DH)), full((1, DH)),
                 full((1, 1)), full((1, 1))]
    in_specs += [full((DH, DH)), full((1, DH)),
                 full((DH, 1)), full((DH, 1)), full((1, 1))]

    outs = [jax.ShapeDtypeStruct((N, DH), F32)]
    outs += [jax.ShapeDtypeStruct((N, LANES), F32) for _ in range(nq)]
    outs += [jax.ShapeDtypeStruct((N, 1), F32),
             jax.ShapeDtypeStruct((N, 1), F32)]
    out_specs = [pl.BlockSpec((rb, DH), lambda i: (i, 0))]
    out_specs += [pl.BlockSpec((rb, LANES), lambda i: (i, 0))
                  for _ in range(nq)]
    out_specs += [pl.BlockSpec((rb, 1), lambda i: (i, 0)),
                  pl.BlockSpec((rb, 1), lambda i: (i, 0))]

    def body(*refs):
        h_r = refs[0]
        hq_refs = refs[1:1 + nq]
        p_refs = refs[1 + nq:1 + nq + NC * nq]
        (g_r, be_r, w0_r, w1_r, w_r, b_r, wai_r, waj_r, ba_r,
         hn_r, *rest) = refs[1 + nq + NC * nq:]
        hqo_refs = rest[:nq]
        a_r, bv_r = rest[nq], rest[nq + 1]
        w0 = w0_r[0, 0]
        w1 = w1_r[0, 0]
        cols = []
        for q in range(nq):
            agg = p_refs[q][0, 0] * w0 + p_refs[nq + q][0, 0] * w1
            cols.append(hq_refs[q][...] + agg)
        o = jnp.maximum(jnp.concatenate(cols, axis=-1), 0.0)
        mu = jnp.mean(o, axis=-1, keepdims=True)
        d = o - mu
        var = jnp.mean(d * d, axis=-1, keepdims=True)
        hn = d * lax.rsqrt(var + 1e-5) * g_r[...] + be_r[...]
        hnew = h_r[...] + hn
        hn_r[...] = hnew
        hl = jnp.dot(hnew, w_r[...], preferred_element_type=F32) + b_r[...]
        for q in range(nq):
            hqo_refs[q][...] = hl[:, q * LANES:(q + 1) * LANES]
        a_r[...] = jnp.dot(hl, wai_r[...],
                           preferred_element_type=F32) + ba_r[...]
        bv_r[...] = jnp.dot(hl, waj_r[...], preferred_element_type=F32)

    return pl.pallas_call(
        body, grid=(nblk,), in_specs=in_specs,
        out_specs=out_specs, out_shape=outs)


def kernel(x, edge_index, edge_weight, Wi, bi,
           W0, b0, Wa0, ba0, g0, be0,
           W1, b1, Wa1, ba1, g1, be1,
           W2, b2, Wa2, ba2, g2, be2,
           Wo1, bo1, Wo2, bo2):
    N, DIN = x.shape
    E = edge_index.shape[1]
    DH = Wi.shape[1]
    nq = DH // LANES
    assert E % CHUNK == 0 and DH % LANES == 0
    np_ = -(-N // (8 * NS)) * (8 * NS)
    subr = np_ // NS
    last_sz = N - (NS - 1) * subr
    assert last_sz > 0 and last_sz % 8 == 0
    rb = 400
    assert N % rb == 0

    row = edge_index[0]
    col = edge_index[1]

    pre0 = _make_pre(N, DIN, DH, rb, True)
    score = _make_score(N, E)
    message = _make_message(N, E, DH, np_, subr, last_sz)
    mid = _make_mid(N, DH, rb)
    post_head = _make_post(N, DH, rb, True)

    layers = [(W0, b0, Wa0, ba0, g0, be0),
              (W1, b1, Wa1, ba1, g1, be1),
              (W2, b2, Wa2, ba2, g2, be2)]

    def edge_phase(a, bv):
        _s, c, m, z = score(row, col, a.reshape(N), bv.reshape(N),
                            edge_weight)
        parts = message(row, col, c, *hqs)
        mr = m.reshape(NC, NS * LANES).max(axis=1)
        M = jnp.maximum(mr[0], mr[1])
        zr = z.reshape(NC, NS * LANES).sum(axis=1)
        e0 = jnp.exp(mr[0] - M)
        e1 = jnp.exp(mr[1] - M)
        Z = e0 * zr[0] + e1 * zr[1]
        return parts, (e0 / Z).reshape(1, 1), (e1 / Z).reshape(1, 1)

    W, b, Wa, ba, g, be = layers[0]
    h, *hqs, a, bv = pre0(x, Wi, bi.reshape(1, DH), W, b.reshape(1, DH),
                          Wa[:DH], Wa[DH:], ba.reshape(1, 1))
    for l in range(2):
        parts, w0, w1 = edge_phase(a, bv)
        Wn, bn, Wan, ban, gn, ben = layers[l + 1]
        args = [h] + hqs + [parts] * (NC * nq) + [
            g.reshape(1, DH), be.reshape(1, DH), w0, w1,
            Wn, bn.reshape(1, DH), Wan[:DH], Wan[DH:], ban.reshape(1, 1)]
        h, *rest = mid(*args)
        hqs = rest[:nq]
        a, bv = rest[nq], rest[nq + 1]
        g, be = gn, ben
    parts, w0, w1 = edge_phase(a, bv)
    args = [h] + hqs + [parts] * (NC * nq) + [
        g.reshape(1, DH), be.reshape(1, DH), w0, w1,
        Wo1, bo1.reshape(1, DH // 2), Wo2, bo2.reshape(1, 1)]
    return post_head(*args)
